# Initial kernel scaffold; baseline (speedup 1.0000x reference)
import functools

import jax
import jax.numpy as jnp
from jax import lax
from jax.experimental import pallas as pl
from jax.experimental.pallas import tpu as pltpu
from jax.experimental.pallas import tpu_sc as plsc

N = 10000
E = 320000
H = 128
NORM = 100.0

NC = 2
NS = 16
NW = NC * NS
EPW = E // NW
CH = 80
NCHUNK = EPW // CH

ROWS_A = 640


def _pre_node(h, W1a, W1b):
    def body(h_ref, wa_ref, wb_ref, a_ref, b_ref):
        hv = h_ref[...]
        a_ref[...] = jnp.dot(hv, wa_ref[...], preferred_element_type=jnp.float32)
        b_ref[...] = jnp.dot(hv, wb_ref[...], preferred_element_type=jnp.float32)

    BN = 2000
    return pl.pallas_call(
        body,
        grid=(N // BN,),
        in_specs=[
            pl.BlockSpec((BN, H), lambda i: (i, 0)),
            pl.BlockSpec((H, H), lambda i: (0, 0)),
            pl.BlockSpec((H, H), lambda i: (0, 0)),
        ],
        out_specs=[
            pl.BlockSpec((BN, H), lambda i: (i, 0)),
            pl.BlockSpec((BN, H), lambda i: (i, 0)),
        ],
        out_shape=[
            jax.ShapeDtypeStruct((N, H), jnp.float32),
            jax.ShapeDtypeStruct((N, H), jnp.float32),
        ],
    )(h, W1a, W1b)


def _sc_gather(A, B, idx3r, idx3c):
    mesh = plsc.VectorSubcoreMesh(core_axis_name="c", subcore_axis_name="s")

    @functools.partial(
        pl.kernel,
        mesh=mesh,
        out_type=[
            jax.ShapeDtypeStruct((E, H), jnp.float32),
            jax.ShapeDtypeStruct((E, H), jnp.float32),
        ],
        scratch_types=[
            pltpu.VMEM((NCHUNK, CH), jnp.int32),
            pltpu.VMEM((NCHUNK, CH), jnp.int32),
            pltpu.VMEM((CH, H), jnp.float32),
            pltpu.VMEM((CH, H), jnp.float32),
            pltpu.SemaphoreType.DMA,
            pltpu.SemaphoreType.DMA,
        ],
    )
    def k(a_hbm, b_hbm, ir_hbm, ic_hbm, p_hbm, q_hbm, ir_v, ic_v, bufa, bufb,
          sema, semb):
        cid = lax.axis_index("c")
        sid = lax.axis_index("s")
        wid = sid * NC + cid
        base = wid * EPW
        pltpu.sync_copy(ir_hbm.at[wid], ir_v)
        pltpu.sync_copy(ic_hbm.at[wid], ic_v)

        def body(j, carry):
            off = base + j * CH
            cpa = pltpu.async_copy(a_hbm.at[ir_v.at[j]], bufa, sema)
            cpb = pltpu.async_copy(b_hbm.at[ic_v.at[j]], bufb, semb)
            cpa.wait()
            pltpu.sync_copy(bufa, p_hbm.at[pl.ds(off, CH)])
            cpb.wait()
            pltpu.sync_copy(bufb, q_hbm.at[pl.ds(off, CH)])
            return carry

        lax.fori_loop(0, NCHUNK, body, 0)

    return k(A, B, idx3r, idx3c)


def _edge_mlp(P, Q, edge_attr, cd4, w1c, b1r, W2, b2r, w3r):
    BE = 1280

    def body(p_ref, q_ref, ea_ref, cd_ref, w1c_ref, b1_ref, w2_ref, b2_ref,
             w3_ref, o_ref):
        s = p_ref[...] + q_ref[...] + ea_ref[...] * w1c_ref[...] + b1_ref[...]
        x1 = jax.nn.silu(s)
        x2 = jax.nn.silu(
            jnp.dot(x1, w2_ref[...], preferred_element_type=jnp.float32)
            + b2_ref[...])
        m = jnp.sum(x2 * w3_ref[...], axis=1, keepdims=True)
        o_ref[...] = cd_ref[...] * m

    return pl.pallas_call(
        body,
        grid=(E // BE,),
        in_specs=[
            pl.BlockSpec((BE, H), lambda i: (i, 0)),
            pl.BlockSpec((BE, H), lambda i: (i, 0)),
            pl.BlockSpec((BE, 1), lambda i: (i, 0)),
            pl.BlockSpec((BE, 4), lambda i: (i, 0)),
            pl.BlockSpec((1, H), lambda i: (0, 0)),
            pl.BlockSpec((1, H), lambda i: (0, 0)),
            pl.BlockSpec((H, H), lambda i: (0, 0)),
            pl.BlockSpec((1, H), lambda i: (0, 0)),
            pl.BlockSpec((1, H), lambda i: (0, 0)),
        ],
        out_specs=pl.BlockSpec((BE, 4), lambda i: (i, 0)),
        out_shape=jax.ShapeDtypeStruct((E, 4), jnp.float32),
    )(P, Q, edge_attr, cd4, w1c, b1r, W2, b2r, w3r)


def _sc_scatter(t4, idx3r, zeros4):
    mesh = plsc.VectorSubcoreMesh(core_axis_name="c", subcore_axis_name="s")

    @functools.partial(
        pl.kernel,
        mesh=mesh,
        out_type=jax.ShapeDtypeStruct((NC, N, 4), jnp.float32),
        scratch_types=[
            pltpu.VMEM((NCHUNK, CH), jnp.int32),
            pltpu.VMEM((EPW, 4), jnp.float32),
            pltpu.VMEM_SHARED((N, 4), jnp.float32),
        ],
    )
    def k(t_hbm, ir_hbm, z_hbm, part_hbm, ir_v, tbuf, shared):
        cid = lax.axis_index("c")
        sid = lax.axis_index("s")
        wid = sid * NC + cid
        base = wid * EPW
        pltpu.sync_copy(ir_hbm.at[wid], ir_v)
        pltpu.sync_copy(t_hbm.at[pl.ds(base, EPW)], tbuf)

        @pl.when(sid == 0)
        def _():
            pltpu.sync_copy(z_hbm, shared)

        plsc.subcore_barrier()

        def body(j, carry):
            pltpu.sync_copy(tbuf.at[pl.ds(j * CH, CH)],
                            shared.at[ir_v.at[j]], add=True)
            return carry

        lax.fori_loop(0, NCHUNK, body, 0)
        plsc.subcore_barrier()

        @pl.when(sid < NS - 1)
        def _():
            pltpu.sync_copy(shared.at[pl.ds(sid * ROWS_A, ROWS_A)],
                            part_hbm.at[cid, pl.ds(sid * ROWS_A, ROWS_A)])

        @pl.when(sid == NS - 1)
        def _():
            last = N - (NS - 1) * ROWS_A
            pltpu.sync_copy(shared.at[pl.ds((NS - 1) * ROWS_A, last)],
                            part_hbm.at[cid, pl.ds((NS - 1) * ROWS_A, last)])

    return k(t4, idx3r, zeros4)


def _final(coord, parts):
    def body(c_ref, p_ref, o_ref):
        acc = p_ref[0] + p_ref[1]
        o_ref[...] = c_ref[...] + acc[:, :3] * (1.0 / NORM)

    return pl.pallas_call(
        body,
        grid=(1,),
        in_specs=[
            pl.BlockSpec((N, 3), lambda i: (0, 0)),
            pl.BlockSpec((NC, N, 4), lambda i: (0, 0, 0)),
        ],
        out_specs=pl.BlockSpec((N, 3), lambda i: (0, 0)),
        out_shape=jax.ShapeDtypeStruct((N, 3), jnp.float32),
    )(coord, parts)


def kernel(h, coord, edge_index, coord_diff, edge_attr, W1, b1, W2, b2, W3):
    row = edge_index[0].astype(jnp.int32)
    col = edge_index[1].astype(jnp.int32)
    W1a = W1[:H]
    W1b = W1[H:2 * H]
    w1c = W1[2 * H:2 * H + 1]

    A, B = _pre_node(h, W1a, W1b)
    idx3r = row.reshape(NW, NCHUNK, CH)
    idx3c = col.reshape(NW, NCHUNK, CH)
    P, Q = _sc_gather(A, B, idx3r, idx3c)
    cd4 = jnp.pad(coord_diff, ((0, 0), (0, 1)))
    t4 = _edge_mlp(P, Q, edge_attr, cd4, w1c, b1.reshape(1, H), W2,
                   b2.reshape(1, H), W3.reshape(1, H))
    zeros4 = jnp.zeros((N, 4), jnp.float32)
    parts = _sc_scatter(t4, idx3r, zeros4)
    return _final(coord, parts)


# speedup vs baseline: 2.6640x; 2.6640x over previous
import functools

import jax
import jax.numpy as jnp
from jax import lax
from jax.experimental import pallas as pl
from jax.experimental.pallas import tpu as pltpu
from jax.experimental.pallas import tpu_sc as plsc

N = 10000
E = 320000
H = 128
NORM = 100.0

NC = 2
NS = 16
NW = NC * NS
EPW = E // NW
CH = 80
NCHUNK = EPW // CH

ES = E // NS
NCHS = ES // CH
RPT = 640


def _pre_node(h, W1a, W1b):
    def body(h_ref, wa_ref, wb_ref, a_ref, b_ref):
        hv = h_ref[...]
        a_ref[...] = jnp.dot(hv, wa_ref[...], preferred_element_type=jnp.float32)
        b_ref[...] = jnp.dot(hv, wb_ref[...], preferred_element_type=jnp.float32)

    BN = 2000
    return pl.pallas_call(
        body,
        grid=(N // BN,),
        in_specs=[
            pl.BlockSpec((BN, H), lambda i: (i, 0)),
            pl.BlockSpec((H, H), lambda i: (0, 0)),
            pl.BlockSpec((H, H), lambda i: (0, 0)),
        ],
        out_specs=[
            pl.BlockSpec((BN, H), lambda i: (i, 0)),
            pl.BlockSpec((BN, H), lambda i: (i, 0)),
        ],
        out_shape=[
            jax.ShapeDtypeStruct((N, H), jnp.float32),
            jax.ShapeDtypeStruct((N, H), jnp.float32),
        ],
    )(h, W1a, W1b)


def _sc_gather(A, B, idx3r, idx3c):
    mesh = plsc.VectorSubcoreMesh(core_axis_name="c", subcore_axis_name="s")

    @functools.partial(
        pl.kernel,
        mesh=mesh,
        out_type=[
            jax.ShapeDtypeStruct((E, H), jnp.float32),
            jax.ShapeDtypeStruct((E, H), jnp.float32),
        ],
        scratch_types=[
            pltpu.VMEM((NCHUNK, CH), jnp.int32),
            pltpu.VMEM((NCHUNK, CH), jnp.int32),
            pltpu.VMEM((CH, H), jnp.float32),
            pltpu.VMEM((CH, H), jnp.float32),
            pltpu.SemaphoreType.DMA,
            pltpu.SemaphoreType.DMA,
        ],
    )
    def k(a_hbm, b_hbm, ir_hbm, ic_hbm, p_hbm, q_hbm, ir_v, ic_v, bufa, bufb,
          sema, semb):
        cid = lax.axis_index("c")
        sid = lax.axis_index("s")
        wid = sid * NC + cid
        base = wid * EPW
        pltpu.sync_copy(ir_hbm.at[wid], ir_v)
        pltpu.sync_copy(ic_hbm.at[wid], ic_v)

        def body(j, carry):
            off = base + j * CH
            cpa = pltpu.async_copy(a_hbm.at[ir_v.at[j]], bufa, sema)
            cpb = pltpu.async_copy(b_hbm.at[ic_v.at[j]], bufb, semb)
            cpa.wait()
            pltpu.sync_copy(bufa, p_hbm.at[pl.ds(off, CH)])
            cpb.wait()
            pltpu.sync_copy(bufb, q_hbm.at[pl.ds(off, CH)])
            return carry

        lax.fori_loop(0, NCHUNK, body, 0)

    return k(A, B, idx3r, idx3c)


def _edge_mlp(P, Q, edge_attr, cdx, cdy, cdz, w1c, b1r, W2, b2r, w3r):
    BE = 512

    def body(p_ref, q_ref, ea_ref, cx_ref, cy_ref, cz_ref, w1c_ref, b1_ref,
             w2_ref, b2_ref, w3_ref, tx_ref, ty_ref, tz_ref):
        s = p_ref[...] + q_ref[...] + ea_ref[...] * w1c_ref[...] + b1_ref[...]
        x1 = jax.nn.silu(s)
        x2 = jax.nn.silu(
            jnp.dot(x1, w2_ref[...], preferred_element_type=jnp.float32)
            + b2_ref[...])
        m = jnp.sum(x2 * w3_ref[...], axis=1)
        tx_ref[...] = cx_ref[...] * m
        ty_ref[...] = cy_ref[...] * m
        tz_ref[...] = cz_ref[...] * m

    return pl.pallas_call(
        body,
        grid=(E // BE,),
        in_specs=[
            pl.BlockSpec((BE, H), lambda i: (i, 0)),
            pl.BlockSpec((BE, H), lambda i: (i, 0)),
            pl.BlockSpec((BE, 1), lambda i: (i, 0)),
            pl.BlockSpec((BE,), lambda i: (i,)),
            pl.BlockSpec((BE,), lambda i: (i,)),
            pl.BlockSpec((BE,), lambda i: (i,)),
            pl.BlockSpec((1, H), lambda i: (0, 0)),
            pl.BlockSpec((1, H), lambda i: (0, 0)),
            pl.BlockSpec((H, H), lambda i: (0, 0)),
            pl.BlockSpec((1, H), lambda i: (0, 0)),
            pl.BlockSpec((1, H), lambda i: (0, 0)),
        ],
        out_specs=[
            pl.BlockSpec((BE,), lambda i: (i,)),
            pl.BlockSpec((BE,), lambda i: (i,)),
            pl.BlockSpec((BE,), lambda i: (i,)),
        ],
        out_shape=[
            jax.ShapeDtypeStruct((E,), jnp.float32),
            jax.ShapeDtypeStruct((E,), jnp.float32),
            jax.ShapeDtypeStruct((E,), jnp.float32),
        ],
    )(P, Q, edge_attr, cdx, cdy, cdz, w1c, b1r, W2, b2r, w3r)


def _sc_scatter_finalize(tx, ty, tz, idx3s, cx, cy, cz):
    mesh = plsc.VectorSubcoreMesh(core_axis_name="c", subcore_axis_name="s")

    @functools.partial(
        pl.kernel,
        mesh=mesh,
        out_type=[
            jax.ShapeDtypeStruct((N,), jnp.float32),
            jax.ShapeDtypeStruct((N,), jnp.float32),
            jax.ShapeDtypeStruct((N,), jnp.float32),
        ],
        scratch_types=[
            pltpu.VMEM((NCHS, CH), jnp.int32),
            pltpu.VMEM((ES,), jnp.float32),
            pltpu.VMEM((ES,), jnp.float32),
            pltpu.VMEM((ES,), jnp.float32),
            pltpu.VMEM((RPT,), jnp.float32),
            pltpu.VMEM((RPT,), jnp.float32),
            pltpu.VMEM((RPT,), jnp.float32),
            pltpu.VMEM_SHARED((N,), jnp.float32),
            pltpu.VMEM_SHARED((N,), jnp.float32),
            pltpu.VMEM_SHARED((N,), jnp.float32),
        ],
    )
    def k(tx_h, ty_h, tz_h, ix_h, cx_h, cy_h, cz_h, ox_h, oy_h, oz_h,
          ix_v, txv, tyv, tzv, avbuf, cbuf, obuf, accx, accy, accz):
        sid = lax.axis_index("s")
        base = sid * ES
        pltpu.sync_copy(ix_h.at[sid], ix_v)
        pltpu.sync_copy(tx_h.at[pl.ds(base, ES)], txv)
        pltpu.sync_copy(ty_h.at[pl.ds(base, ES)], tyv)
        pltpu.sync_copy(tz_h.at[pl.ds(base, ES)], tzv)

        def zb(i, carry):
            avbuf[pl.ds(i * 16, 16)] = jnp.zeros((16,), jnp.float32)
            return carry

        lax.fori_loop(0, RPT // 16, zb, 0)
        row0 = sid * RPT

        def zero_acc(nrows):
            pltpu.sync_copy(avbuf.at[pl.ds(0, nrows)], accx.at[pl.ds(row0, nrows)])
            pltpu.sync_copy(avbuf.at[pl.ds(0, nrows)], accy.at[pl.ds(row0, nrows)])
            pltpu.sync_copy(avbuf.at[pl.ds(0, nrows)], accz.at[pl.ds(row0, nrows)])

        @pl.when(sid < NS - 1)
        def _():
            zero_acc(RPT)

        @pl.when(sid == NS - 1)
        def _():
            zero_acc(N - (NS - 1) * RPT)

        plsc.subcore_barrier()

        def body(j, carry):
            src = pl.ds(j * CH, CH)
            pltpu.sync_copy(txv.at[src], accx.at[ix_v.at[j]], add=True)
            pltpu.sync_copy(tyv.at[src], accy.at[ix_v.at[j]], add=True)
            pltpu.sync_copy(tzv.at[src], accz.at[ix_v.at[j]], add=True)
            return carry

        lax.fori_loop(0, NCHS, body, 0)
        plsc.subcore_barrier()

        def fin(acc, c_h, o_h, nrows):
            pltpu.sync_copy(acc.at[pl.ds(row0, nrows)], avbuf.at[pl.ds(0, nrows)])
            pltpu.sync_copy(c_h.at[pl.ds(row0, nrows)], cbuf.at[pl.ds(0, nrows)])

            def fb(i, carry):
                sl = pl.ds(i * 16, 16)
                obuf[sl] = cbuf[sl] + avbuf[sl] * (1.0 / NORM)
                return carry

            lax.fori_loop(0, nrows // 16, fb, 0)
            pltpu.sync_copy(obuf.at[pl.ds(0, nrows)], o_h.at[pl.ds(row0, nrows)])

        def fin_all(nrows):
            fin(accx, cx_h, ox_h, nrows)
            fin(accy, cy_h, oy_h, nrows)
            fin(accz, cz_h, oz_h, nrows)

        @pl.when(sid < NS - 1)
        def _():
            fin_all(RPT)

        @pl.when(sid == NS - 1)
        def _():
            fin_all(N - (NS - 1) * RPT)

    return k(tx, ty, tz, idx3s, cx, cy, cz)


def kernel(h, coord, edge_index, coord_diff, edge_attr, W1, b1, W2, b2, W3):
    row = edge_index[0].astype(jnp.int32)
    col = edge_index[1].astype(jnp.int32)
    W1a = W1[:H]
    W1b = W1[H:2 * H]
    w1c = W1[2 * H:2 * H + 1]

    A, B = _pre_node(h, W1a, W1b)
    idx3r = row.reshape(NW, NCHUNK, CH)
    idx3c = col.reshape(NW, NCHUNK, CH)
    P, Q = _sc_gather(A, B, idx3r, idx3c)

    cdx = coord_diff[:, 0]
    cdy = coord_diff[:, 1]
    cdz = coord_diff[:, 2]
    tx, ty, tz = _edge_mlp(P, Q, edge_attr, cdx, cdy, cdz, w1c,
                           b1.reshape(1, H), W2, b2.reshape(1, H),
                           W3.reshape(1, H))

    idx3s = row.reshape(NS, NCHS, CH)
    cx = coord[:, 0]
    cy = coord[:, 1]
    cz = coord[:, 2]
    ox, oy, oz = _sc_scatter_finalize(tx, ty, tz, idx3s, cx, cy, cz)
    return jnp.stack([ox, oy, oz], axis=1)
